# baseline (speedup 1.0000x reference)
import functools

import jax
import jax.numpy as jnp
from jax import lax
from jax.experimental import pallas as pl
from jax.experimental.pallas import tpu as pltpu
from jax.experimental.pallas import tpu_sc as plsc

BATCH = 16384
VOCAB = 1000000
EMB_DIM = 2
N_CLASSES = 100

_NC = 2
_NS = 16
_NW = _NC * _NS
_CHUNK = 128
_PW = BATCH // _NW
_KPW = _PW // _CHUNK
_TBL_D = 8
_TBL_V = VOCAB * EMB_DIM // _TBL_D


def _sc_gather(table8, ridx, col0):
    mesh = plsc.VectorSubcoreMesh(core_axis_name="c", subcore_axis_name="s")

    @functools.partial(
        pl.kernel,
        out_type=jax.ShapeDtypeStruct((EMB_DIM, BATCH), jnp.float32),
        mesh=mesh,
        scratch_types=[
            pltpu.VMEM((_KPW, _CHUNK), jnp.int32),
            pltpu.VMEM((_PW,), jnp.int32),
            pltpu.VMEM((_PW, _TBL_D), jnp.float32),
            pltpu.VMEM((_PW,), jnp.float32),
            pltpu.VMEM((_PW,), jnp.float32),
            pltpu.SemaphoreType.DMA,
        ],
        compiler_params=pltpu.CompilerParams(
            use_tc_tiling_on_sc=False, needs_layout_passes=False
        ),
    )
    def gather_kernel(tbl, ridx_h, col_h, out_h, ridx_v, col_v, rows_v,
                      e0_v, e1_v, sem):
        wid = lax.axis_index("s") * _NC + lax.axis_index("c")
        base = wid * _PW
        pltpu.sync_copy(ridx_h.at[wid], ridx_v)
        pltpu.sync_copy(col_h.at[wid], col_v)
        copies = [
            pltpu.async_copy(
                tbl.at[ridx_v.at[j]],
                rows_v.at[pl.ds(j * _CHUNK, _CHUNK)],
                sem,
            )
            for j in range(_KPW)
        ]
        for c in copies:
            c.wait()
        for m in range(_PW // 16):
            rid = lax.iota(jnp.int32, 16) + (m * 16)
            cols = col_v[pl.ds(m * 16, 16)]
            e0_v[pl.ds(m * 16, 16)] = plsc.load_gather(rows_v, [rid, cols])
            e1_v[pl.ds(m * 16, 16)] = plsc.load_gather(rows_v, [rid, cols + 1])
        pltpu.sync_copy(e0_v, out_h.at[0, pl.ds(base, _PW)])
        pltpu.sync_copy(e1_v, out_h.at[1, pl.ds(base, _PW)])

    return gather_kernel(table8, ridx, col0)


def _tc_dense_softmax(eT, wt, b2):
    rows = 2048
    grid = BATCH // rows

    def body(et_ref, wt_ref, b_ref, out_ref):
        logits = lax.dot_general(
            et_ref[...], wt_ref[...],
            (((0,), (0,)), ((), ())),
            preferred_element_type=jnp.float32,
        ) + b_ref[...]
        m = jnp.max(logits, axis=1, keepdims=True)
        p = jnp.exp(logits - m)
        out_ref[...] = p / jnp.sum(p, axis=1, keepdims=True)

    return pl.pallas_call(
        body,
        grid=(grid,),
        in_specs=[
            pl.BlockSpec((EMB_DIM, rows), lambda i: (0, i)),
            pl.BlockSpec((EMB_DIM, N_CLASSES), lambda i: (0, 0)),
            pl.BlockSpec((1, N_CLASSES), lambda i: (0, 0)),
        ],
        out_specs=pl.BlockSpec((rows, N_CLASSES), lambda i: (i, 0)),
        out_shape=jax.ShapeDtypeStruct((BATCH, N_CLASSES), jnp.float32),
    )(eT, wt, b2)


@jax.jit
def kernel(x, emb, W, b):
    x32 = x.astype(jnp.int32)
    ridx = (x32 >> 2).reshape(_NW, _KPW, _CHUNK)
    col0 = ((x32 & 3) << 1).reshape(_NW, _PW)
    table8 = emb.reshape(_TBL_V, _TBL_D)
    eT = _sc_gather(table8, ridx, col0)
    return _tc_dense_softmax(eT, W.T, b.reshape(1, N_CLASSES))


# speedup vs baseline: 4.9477x; 4.9477x over previous
import functools

import jax
import jax.numpy as jnp
from jax import lax
from jax.experimental import pallas as pl
from jax.experimental.pallas import tpu as pltpu
from jax.experimental.pallas import tpu_sc as plsc

BATCH = 16384
VOCAB = 1000000
EMB_DIM = 2
N_CLASSES = 100

_NC = 2
_NS = 16
_NW = _NC * _NS
_PW = BATCH // _NW
_NG = _PW // 16


def _sc_gather(table, idx):
    mesh = plsc.VectorSubcoreMesh(core_axis_name="c", subcore_axis_name="s")

    @functools.partial(
        pl.kernel,
        out_type=jax.ShapeDtypeStruct((EMB_DIM, BATCH), jnp.float32),
        mesh=mesh,
        scratch_types=[
            pltpu.VMEM((_PW,), jnp.int32),
            pltpu.VMEM((_PW, EMB_DIM), jnp.float32),
            pltpu.VMEM((_PW,), jnp.float32),
            pltpu.VMEM((_PW,), jnp.float32),
            pltpu.SemaphoreType.DMA,
        ],
        compiler_params=pltpu.CompilerParams(
            use_tc_tiling_on_sc=True, needs_layout_passes=False
        ),
    )
    def gather_kernel(tbl, idx_h, out_h, idx_v, buf_v, e0_v, e1_v, sem):
        wid = lax.axis_index("s") * _NC + lax.axis_index("c")
        base = wid * _PW
        pltpu.sync_copy(idx_h.at[wid], idx_v)
        lanes = lax.iota(jnp.int32, 16)

        def fire_group(m):
            v = idx_v[pl.ds(m * 16, 16)]
            for t in range(16):
                i0 = lax.reduce_sum(jnp.where(lanes == t, v, 0), axes=(0,))
                pltpu.async_copy(
                    tbl.at[pl.ds(i0, 1)], buf_v.at[pl.ds(m * 16 + t, 1)], sem
                )

        def drain_group(m):
            for t in range(16):
                pltpu.make_async_copy(
                    tbl.at[pl.ds(0, 1)], buf_v.at[pl.ds(m * 16 + t, 1)], sem
                ).wait()

        def body(m, carry):
            fire_group(m)

            @pl.when(m > 0)
            def _():
                drain_group(m - 1)

            return carry

        lax.fori_loop(0, _NG, body, 0)
        drain_group(_NG - 1)

        zeros = jnp.zeros((16,), jnp.int32)
        ones = zeros + 1
        for m in range(_NG):
            rid = lanes + (m * 16)
            e0_v[pl.ds(m * 16, 16)] = plsc.load_gather(buf_v, [rid, zeros])
            e1_v[pl.ds(m * 16, 16)] = plsc.load_gather(buf_v, [rid, ones])
        pltpu.sync_copy(e0_v, out_h.at[0, pl.ds(base, _PW)])
        pltpu.sync_copy(e1_v, out_h.at[1, pl.ds(base, _PW)])

    return gather_kernel(table, idx)


def _tc_dense_softmax(eT, wt, b2):
    rows = 2048
    grid = BATCH // rows

    def body(et_ref, wt_ref, b_ref, out_ref):
        logits = lax.dot_general(
            et_ref[...], wt_ref[...],
            (((0,), (0,)), ((), ())),
            preferred_element_type=jnp.float32,
        ) + b_ref[...]
        m = jnp.max(logits, axis=1, keepdims=True)
        p = jnp.exp(logits - m)
        out_ref[...] = p / jnp.sum(p, axis=1, keepdims=True)

    return pl.pallas_call(
        body,
        grid=(grid,),
        in_specs=[
            pl.BlockSpec((EMB_DIM, rows), lambda i: (0, i)),
            pl.BlockSpec((EMB_DIM, N_CLASSES), lambda i: (0, 0)),
            pl.BlockSpec((1, N_CLASSES), lambda i: (0, 0)),
        ],
        out_specs=pl.BlockSpec((rows, N_CLASSES), lambda i: (i, 0)),
        out_shape=jax.ShapeDtypeStruct((BATCH, N_CLASSES), jnp.float32),
    )(eT, wt, b2)


@jax.jit
def kernel(x, emb, W, b):
    idx = x.astype(jnp.int32).reshape(_NW, _PW)
    eT = _sc_gather(emb, idx)
    return _tc_dense_softmax(eT, W.T, b.reshape(1, N_CLASSES))
